# baseline (speedup 1.0000x reference)
import functools

import jax
import jax.numpy as jnp
import numpy as np
from jax import lax
from jax.experimental import pallas as pl
from jax.experimental.pallas import tpu as pltpu
from jax.experimental.pallas import tpu_sc as plsc

_B, _T, _C = 64, 2048, 128


def _static_crop_params(B, T):
    rng = np.random.RandomState(0)
    crop_l = int(rng.randint(2, T + 1))
    crop_left = int(rng.randint(T - crop_l + 1))
    crop_right = crop_left + crop_l
    crop_eleft = int(rng.randint(crop_left + 1))
    crop_eright = int(rng.randint(crop_right, T + 1))
    crop_offset = rng.randint(-crop_eleft, T - crop_eright + 1, size=B)
    return crop_l, crop_left, crop_right, crop_eleft, crop_eright, crop_offset


(_CROP_L, _CROP_LEFT, _CROP_RIGHT, _CROP_ELEFT, _CROP_ERIGHT,
 _CROP_OFFSET) = _static_crop_params(_B, _T)
_W1 = _CROP_RIGHT - _CROP_ELEFT
_W2 = _CROP_ERIGHT - _CROP_LEFT
_S1 = [int(v) for v in (_CROP_OFFSET + _CROP_ELEFT)]
_S2 = [int(v) for v in (_CROP_OFFSET + _CROP_LEFT)]

_NUM_CORES = 2
_NUM_SUBCORES = 16
_NW = _NUM_CORES * _NUM_SUBCORES
_ROWS_PER_W = _B // _NW


@functools.partial(
    pl.kernel,
    out_type=(
        jax.ShapeDtypeStruct((_B * _W1 * _C,), jnp.float32),
        jax.ShapeDtypeStruct((_B * _W2 * _C,), jnp.float32),
    ),
    mesh=plsc.VectorSubcoreMesh(
        core_axis_name="c", subcore_axis_name="s",
        num_cores=_NUM_CORES, num_subcores=_NUM_SUBCORES),
    scratch_types=[pltpu.SemaphoreType.DMA],
)
def _crop_sc(x_hbm, out1_hbm, out2_hbm, sem):
    wid = lax.axis_index("s") * _NUM_CORES + lax.axis_index("c")
    for w in range(_NW):
        @pl.when(wid == w)
        def _copies(w=w):
            copies = []
            for j in range(_ROWS_PER_W):
                b = w * _ROWS_PER_W + j
                copies.append(pltpu.make_async_copy(
                    x_hbm.at[pl.ds((b * _T + _S1[b]) * _C, _W1 * _C)],
                    out1_hbm.at[pl.ds(b * _W1 * _C, _W1 * _C)], sem))
                copies.append(pltpu.make_async_copy(
                    x_hbm.at[pl.ds((b * _T + _S2[b]) * _C, _W2 * _C)],
                    out2_hbm.at[pl.ds(b * _W2 * _C, _W2 * _C)], sem))
            for c in copies:
                c.start()
            for c in copies:
                c.wait()


def kernel(x):
    s1, s2 = _crop_sc(x.reshape(-1))
    return (s1.reshape(_B, _W1, _C), s2.reshape(_B, _W2, _C))


# speedup vs baseline: 11.6166x; 11.6166x over previous
import functools

import jax
import jax.numpy as jnp
import numpy as np
from jax import lax
from jax.experimental import pallas as pl
from jax.experimental.pallas import tpu as pltpu
from jax.experimental.pallas import tpu_sc as plsc

_B, _T, _C = 64, 2048, 128


def _static_crop_params(B, T):
    rng = np.random.RandomState(0)
    crop_l = int(rng.randint(2, T + 1))
    crop_left = int(rng.randint(T - crop_l + 1))
    crop_right = crop_left + crop_l
    crop_eleft = int(rng.randint(crop_left + 1))
    crop_eright = int(rng.randint(crop_right, T + 1))
    crop_offset = rng.randint(-crop_eleft, T - crop_eright + 1, size=B)
    return crop_l, crop_left, crop_right, crop_eleft, crop_eright, crop_offset


(_CROP_L, _CROP_LEFT, _CROP_RIGHT, _CROP_ELEFT, _CROP_ERIGHT,
 _CROP_OFFSET) = _static_crop_params(_B, _T)
_W1 = _CROP_RIGHT - _CROP_ELEFT
_W2 = _CROP_ERIGHT - _CROP_LEFT
_S1 = [int(v) for v in (_CROP_OFFSET + _CROP_ELEFT)]
_S2 = [int(v) for v in (_CROP_OFFSET + _CROP_LEFT)]

_NUM_CORES = 2
_NUM_SUBCORES = 16
_NW = _NUM_CORES * _NUM_SUBCORES
_ROWS_PER_W = _B // _NW


_CHUNK = 64512


def _subcore_tasks(w):
    tasks = []
    for j in range(_ROWS_PER_W):
        b = w * _ROWS_PER_W + j
        tasks.append(((b * _T + _S1[b]) * _C, 0, b * _W1 * _C, _W1 * _C))
        tasks.append(((b * _T + _S2[b]) * _C, 1, b * _W2 * _C, _W2 * _C))
    chunks = []
    for src, oi, dst, ln in tasks:
        done = 0
        while done < ln:
            n = min(_CHUNK, ln - done)
            chunks.append((src + done, oi, dst + done, n))
            done += n
    return chunks


@functools.partial(
    pl.kernel,
    out_type=(
        jax.ShapeDtypeStruct((_B * _W1 * _C,), jnp.float32),
        jax.ShapeDtypeStruct((_B * _W2 * _C,), jnp.float32),
    ),
    mesh=plsc.VectorSubcoreMesh(
        core_axis_name="c", subcore_axis_name="s",
        num_cores=_NUM_CORES, num_subcores=_NUM_SUBCORES),
    scratch_types=[
        pltpu.VMEM((_CHUNK,), jnp.float32),
        pltpu.VMEM((_CHUNK,), jnp.float32),
        pltpu.SemaphoreType.DMA,
        pltpu.SemaphoreType.DMA,
    ],
)
def _crop_sc(x_hbm, out1_hbm, out2_hbm, buf0, buf1, in_sem, out_sem):
    wid = lax.axis_index("s") * _NUM_CORES + lax.axis_index("c")
    for w in range(_NW):
        @pl.when(wid == w)
        def _copies(w=w):
            bufs = (buf0, buf1)
            outs = (out1_hbm, out2_hbm)
            chunks = _subcore_tasks(w)
            n = len(chunks)
            cin, cout = [], []
            for i, (src, oi, dst, ln) in enumerate(chunks):
                buf = bufs[i % 2].at[pl.ds(0, ln)]
                cin.append(pltpu.make_async_copy(
                    x_hbm.at[pl.ds(src, ln)], buf, in_sem))
                cout.append(pltpu.make_async_copy(
                    buf, outs[oi].at[pl.ds(dst, ln)], out_sem))
            cin[0].start()
            for i in range(n):
                if i > 0:
                    cout[i - 1].wait()
                if i + 1 < n:
                    cin[i + 1].start()
                cin[i].wait()
                cout[i].start()
            cout[n - 1].wait()


def kernel(x):
    s1, s2 = _crop_sc(x.reshape(-1))
    return (s1.reshape(_B, _W1, _C), s2.reshape(_B, _W2, _C))


# speedup vs baseline: 12.1019x; 1.0418x over previous
import functools

import jax
import jax.numpy as jnp
import numpy as np
from jax import lax
from jax.experimental import pallas as pl
from jax.experimental.pallas import tpu as pltpu
from jax.experimental.pallas import tpu_sc as plsc

_B, _T, _C = 64, 2048, 128


def _static_crop_params(B, T):
    rng = np.random.RandomState(0)
    crop_l = int(rng.randint(2, T + 1))
    crop_left = int(rng.randint(T - crop_l + 1))
    crop_right = crop_left + crop_l
    crop_eleft = int(rng.randint(crop_left + 1))
    crop_eright = int(rng.randint(crop_right, T + 1))
    crop_offset = rng.randint(-crop_eleft, T - crop_eright + 1, size=B)
    return crop_l, crop_left, crop_right, crop_eleft, crop_eright, crop_offset


(_CROP_L, _CROP_LEFT, _CROP_RIGHT, _CROP_ELEFT, _CROP_ERIGHT,
 _CROP_OFFSET) = _static_crop_params(_B, _T)
_W1 = _CROP_RIGHT - _CROP_ELEFT
_W2 = _CROP_ERIGHT - _CROP_LEFT
_S1 = [int(v) for v in (_CROP_OFFSET + _CROP_ELEFT)]
_S2 = [int(v) for v in (_CROP_OFFSET + _CROP_LEFT)]

_NUM_CORES = 2
_NUM_SUBCORES = 16
_NW = _NUM_CORES * _NUM_SUBCORES
_ROWS_PER_W = _B // _NW


_CHUNK = 64512

_D21 = _CROP_LEFT - _CROP_ELEFT
_WU = _D21 + _W2
assert 0 <= _D21 <= _W1 <= _WU


def _subcore_tasks(w):
    chunks = []
    for j in range(_ROWS_PER_W):
        b = w * _ROWS_PER_W + j
        base = (b * _T + _S1[b]) * _C
        a = 0
        while a < _WU:
            rows = min(_CHUNK // _C, _WU - a)
            outs = []
            if a < _W1:
                p = min(rows, _W1 - a)
                outs.append((0, 0, (b * _W1 + a) * _C, p * _C))
            if a + rows > _D21:
                lo = max(a, _D21)
                outs.append((1, (lo - a) * _C, (b * _W2 + lo - _D21) * _C,
                             (a + rows - lo) * _C))
            chunks.append((base + a * _C, rows * _C, outs))
            a += rows
    return chunks


@functools.partial(
    pl.kernel,
    out_type=(
        jax.ShapeDtypeStruct((_B * _W1 * _C,), jnp.float32),
        jax.ShapeDtypeStruct((_B * _W2 * _C,), jnp.float32),
    ),
    mesh=plsc.VectorSubcoreMesh(
        core_axis_name="c", subcore_axis_name="s",
        num_cores=_NUM_CORES, num_subcores=_NUM_SUBCORES),
    scratch_types=[
        pltpu.VMEM((_CHUNK,), jnp.float32),
        pltpu.VMEM((_CHUNK,), jnp.float32),
        pltpu.SemaphoreType.DMA,
        pltpu.SemaphoreType.DMA,
    ],
)
def _crop_sc(x_hbm, out1_hbm, out2_hbm, buf0, buf1, in_sem, out_sem):
    wid = lax.axis_index("s") * _NUM_CORES + lax.axis_index("c")
    for w in range(_NW):
        @pl.when(wid == w)
        def _copies(w=w):
            bufs = (buf0, buf1)
            out_refs = (out1_hbm, out2_hbm)
            chunks = _subcore_tasks(w)
            n = len(chunks)
            cin, cout = [], []
            for i, (src, ln, outs) in enumerate(chunks):
                buf = bufs[i % 2]
                cin.append(pltpu.make_async_copy(
                    x_hbm.at[pl.ds(src, ln)], buf.at[pl.ds(0, ln)], in_sem))
                cout.append([pltpu.make_async_copy(
                    buf.at[pl.ds(boff, pln)],
                    out_refs[oi].at[pl.ds(dst, pln)], out_sem)
                    for oi, boff, dst, pln in outs])
            cin[0].start()
            for i in range(n):
                if i > 0:
                    for c in cout[i - 1]:
                        c.wait()
                if i + 1 < n:
                    cin[i + 1].start()
                cin[i].wait()
                for c in cout[i]:
                    c.start()
            for c in cout[n - 1]:
                c.wait()


def kernel(x):
    s1, s2 = _crop_sc(x.reshape(-1))
    return (s1.reshape(_B, _W1, _C), s2.reshape(_B, _W2, _C))


# speedup vs baseline: 17.2775x; 1.4277x over previous
import functools

import jax
import jax.numpy as jnp
import numpy as np
from jax import lax
from jax.experimental import pallas as pl
from jax.experimental.pallas import tpu as pltpu
from jax.experimental.pallas import tpu_sc as plsc

_B, _T, _C = 64, 2048, 128


def _static_crop_params(B, T):
    rng = np.random.RandomState(0)
    crop_l = int(rng.randint(2, T + 1))
    crop_left = int(rng.randint(T - crop_l + 1))
    crop_right = crop_left + crop_l
    crop_eleft = int(rng.randint(crop_left + 1))
    crop_eright = int(rng.randint(crop_right, T + 1))
    crop_offset = rng.randint(-crop_eleft, T - crop_eright + 1, size=B)
    return crop_l, crop_left, crop_right, crop_eleft, crop_eright, crop_offset


(_CROP_L, _CROP_LEFT, _CROP_RIGHT, _CROP_ELEFT, _CROP_ERIGHT,
 _CROP_OFFSET) = _static_crop_params(_B, _T)
_W1 = _CROP_RIGHT - _CROP_ELEFT
_W2 = _CROP_ERIGHT - _CROP_LEFT
_S1 = [int(v) for v in (_CROP_OFFSET + _CROP_ELEFT)]
_S2 = [int(v) for v in (_CROP_OFFSET + _CROP_LEFT)]

_D21 = _CROP_LEFT - _CROP_ELEFT
_WU = _D21 + _W2
assert 0 <= _D21 <= _W1 <= _WU

_NUM_CORES = 2
_NUM_SUBCORES = 16
_NW = _NUM_CORES * _NUM_SUBCORES
_ROWS_PER_W = _B // _NW

_CH = 504


def _row_plan(b):
    s1 = _S1[b]
    base = s1 - (s1 % 8)
    end = s1 + _WU
    step = _CH - 8
    chunks = []
    while base + len(chunks) * step < end:
        c0 = base + len(chunks) * step
        rows = min(_CH, end - c0)
        rows += -rows % 8
        assert c0 + rows <= _T
        chunks.append((c0, rows, []))
    for oi, (s, W) in enumerate(((s1, _W1), (s1 + _D21, _W2))):
        d = 0
        while d < W:
            k = (s + d - base) // step
            c0, rows, pieces = chunks[k]
            d1 = min(W, c0 + rows - s)
            if d1 < W:
                d1 -= d1 % 8
            assert d1 > d
            prow = d1 - d
            prow += -prow % 8
            pieces.append((oi, s + d - c0, d, prow))
            d = d1
    return chunks


@functools.partial(
    pl.kernel,
    out_type=(
        jax.ShapeDtypeStruct((_B, _W1, _C), jnp.float32),
        jax.ShapeDtypeStruct((_B, _W2, _C), jnp.float32),
    ),
    mesh=plsc.VectorSubcoreMesh(
        core_axis_name="c", subcore_axis_name="s",
        num_cores=_NUM_CORES, num_subcores=_NUM_SUBCORES),
    scratch_types=[
        pltpu.VMEM((_CH, _C), jnp.float32),
        pltpu.VMEM((_CH, _C), jnp.float32),
        pltpu.SemaphoreType.DMA,
        pltpu.SemaphoreType.DMA,
    ],
)
def _crop_sc(x_hbm, out1_hbm, out2_hbm, buf0, buf1, in_sem, out_sem):
    wid = lax.axis_index("s") * _NUM_CORES + lax.axis_index("c")
    for w in range(_NW):
        @pl.when(wid == w)
        def _copies(w=w):
            bufs = (buf0, buf1)
            out_refs = (out1_hbm, out2_hbm)
            cin, cout = [], []
            for j in range(_ROWS_PER_W):
                b = w * _ROWS_PER_W + j
                for src_row, rows, pieces in _row_plan(b):
                    buf = bufs[len(cin) % 2]
                    cin.append(pltpu.make_async_copy(
                        x_hbm.at[b, pl.ds(src_row, rows)],
                        buf.at[pl.ds(0, rows)], in_sem))
                    outs = []
                    for oi, vrow, dst, prow in pieces:
                        lim = (_W1 if oi == 0 else _W2)
                        if dst + prow <= lim:
                            dst_idx = pl.ds(dst, prow)
                        else:
                            dst_idx = pl.ds(
                                pl.multiple_of(jnp.int32(dst), 8), prow)
                        outs.append(pltpu.make_async_copy(
                            buf.at[pl.ds(vrow, prow)],
                            out_refs[oi].at[b, dst_idx], out_sem))
                    cout.append(outs)
            n = len(cin)
            cin[0].start()
            for i in range(n):
                if i > 0:
                    for c in cout[i - 1]:
                        c.wait()
                if i + 1 < n:
                    cin[i + 1].start()
                cin[i].wait()
                for c in cout[i]:
                    c.start()
            for c in cout[n - 1]:
                c.wait()


def kernel(x):
    return _crop_sc(x)


# speedup vs baseline: 36.3932x; 2.1064x over previous
import functools

import jax
import jax.numpy as jnp
import numpy as np
from jax import lax
from jax.experimental import pallas as pl
from jax.experimental.pallas import tpu as pltpu
from jax.experimental.pallas import tpu_sc as plsc

_B, _T, _C = 64, 2048, 128


def _static_crop_params(B, T):
    rng = np.random.RandomState(0)
    crop_l = int(rng.randint(2, T + 1))
    crop_left = int(rng.randint(T - crop_l + 1))
    crop_right = crop_left + crop_l
    crop_eleft = int(rng.randint(crop_left + 1))
    crop_eright = int(rng.randint(crop_right, T + 1))
    crop_offset = rng.randint(-crop_eleft, T - crop_eright + 1, size=B)
    return crop_l, crop_left, crop_right, crop_eleft, crop_eright, crop_offset


(_CROP_L, _CROP_LEFT, _CROP_RIGHT, _CROP_ELEFT, _CROP_ERIGHT,
 _CROP_OFFSET) = _static_crop_params(_B, _T)
_W1 = _CROP_RIGHT - _CROP_ELEFT
_W2 = _CROP_ERIGHT - _CROP_LEFT
_S1 = [int(v) for v in (_CROP_OFFSET + _CROP_ELEFT)]
_S2 = [int(v) for v in (_CROP_OFFSET + _CROP_LEFT)]

_D21 = _CROP_LEFT - _CROP_ELEFT
_WU = _D21 + _W2
assert 0 <= _D21 <= _W1 <= _WU

_NUM_CORES = 2
_NUM_SUBCORES = 16
_NW = _NUM_CORES * _NUM_SUBCORES
_ROWS_PER_W = _B // _NW

_NCH = 4
_CHR = -(-_WU // _NCH)


def _row_plan(b):
    s1 = _S1[b]
    chunks = []
    for a in range(0, _WU, _CHR):
        rows = min(_CHR, _WU - a)
        pieces = []
        if a < _W1:
            pieces.append((0, 0, a, min(rows, _W1 - a)))
        if a + rows > _D21:
            lo = max(a, _D21)
            pieces.append((1, lo - a, lo - _D21, a + rows - lo))
        chunks.append((s1 + a, rows, pieces))
    return chunks


@functools.partial(
    pl.kernel,
    out_type=(
        jax.ShapeDtypeStruct((_W1, _B, _C), jnp.float32),
        jax.ShapeDtypeStruct((_W2, _B, _C), jnp.float32),
    ),
    mesh=plsc.VectorSubcoreMesh(
        core_axis_name="c", subcore_axis_name="s",
        num_cores=_NUM_CORES, num_subcores=_NUM_SUBCORES),
    scratch_types=[
        pltpu.VMEM((_CHR, _C), jnp.float32),
        pltpu.VMEM((_CHR, _C), jnp.float32),
        pltpu.SemaphoreType.DMA,
        pltpu.SemaphoreType.DMA,
    ],
    compiler_params=pltpu.CompilerParams(use_tc_tiling_on_sc=False),
)
def _crop_sc(x_hbm, out1_hbm, out2_hbm, buf0, buf1, in_sem, out_sem):
    wid = lax.axis_index("s") * _NUM_CORES + lax.axis_index("c")
    for w in range(_NW):
        @pl.when(wid == w)
        def _copies(w=w):
            bufs = (buf0, buf1)
            out_refs = (out1_hbm, out2_hbm)
            cin, cout = [], []
            for j in range(_ROWS_PER_W):
                b = w * _ROWS_PER_W + j
                for src_row, rows, pieces in _row_plan(b):
                    buf = bufs[len(cin) % 2]
                    cin.append(pltpu.make_async_copy(
                        x_hbm.at[b, pl.ds(src_row, rows)],
                        buf.at[pl.ds(0, rows)], in_sem))
                    cout.append([pltpu.make_async_copy(
                        buf.at[pl.ds(vrow, prow)],
                        out_refs[oi].at[pl.ds(dst, prow), b], out_sem)
                        for oi, vrow, dst, prow in pieces])
            n = len(cin)
            cin[0].start()
            for i in range(n):
                if i > 0:
                    for c in cout[i - 1]:
                        c.wait()
                if i + 1 < n:
                    cin[i + 1].start()
                cin[i].wait()
                for c in cout[i]:
                    c.start()
            for c in cout[n - 1]:
                c.wait()


def kernel(x):
    t1, t2 = _crop_sc(x)
    return (jnp.transpose(t1, (1, 0, 2)), jnp.transpose(t2, (1, 0, 2)))
